# baseline (speedup 1.0000x reference)
import jax
import jax.numpy as jnp
from jax import lax
from jax.experimental import pallas as pl
from jax.experimental.pallas import tpu as pltpu
from jax.experimental.pallas import tpu_sc as plsc

N = 10000
E = 320000
C = 128
NP = 10240
NC = 2
NS = 16
NW = NC * NS
RPT = NP // NW
CH = 80
EB = 20
NB = E // (EB * C)
CAPL = 12800


def _dense_body(bm_ref, emb_ref, wt_ref, attm_ref, attem_ref, x_ref, s_ref):
    x = jnp.dot(bm_ref[...], wt_ref[...], preferred_element_type=jnp.float32)
    x_ref[...] = x
    s_ref[...] = (
        jnp.dot(x, attm_ref[...], preferred_element_type=jnp.float32)
        + jnp.dot(emb_ref[...], attem_ref[...], preferred_element_type=jnp.float32)
    )


def _sc_body(src_hbm, dst_hbm, si_hbm, sj_hbm, x_hbm, out_hbm, den_hbm,
             seb0, deb0, seb1, deb1, si_v, sj_v, srcl_v, rl_v,
             g0, g1, acc_v, den_v, semA, semB, semG0, semG1):
    c = lax.axis_index("c")
    s = lax.axis_index("s")
    wid = s * NC + c
    lo = wid * RPT

    zv = jnp.zeros((16,), jnp.float32)
    ziv = jnp.zeros((16,), jnp.int32)
    lov = jnp.full((16,), lo, jnp.int32)

    def _zacc(i, carry):
        for k in range(C // 16):
            acc_v[i, pl.ds(k * 16, 16)] = zv
        return carry

    lax.fori_loop(0, RPT, _zacc, 0)
    for k in range((RPT + 32) // 16):
        den_v[pl.ds(k * 16, 16)] = zv

    def _zloc(i, carry):
        srcl_v[pl.ds(i * 16, 16)] = lov
        rl_v[pl.ds(i * 16, 16)] = ziv
        return carry

    lax.fori_loop(0, CAPL // 16, _zloc, 0)

    pltpu.sync_copy(si_hbm, si_v)
    pltpu.sync_copy(sj_hbm, sj_v)
    _prof_scan = jax.named_scope("sc_scan")
    _prof_scan.__enter__()

    def _scan_buf(seb, deb, off):
        def _row(i, off):
            for k in range(C // 16):
                s16 = seb[i, pl.ds(k * 16, 16)]
                d16 = deb[i, pl.ds(k * 16, 16)]
                r16 = d16 - lo
                m = (d16 >= lo) & (d16 < lo + RPT)
                plsc.store_compressed(srcl_v.at[pl.ds(off, 16)], s16, mask=m)
                plsc.store_compressed(rl_v.at[pl.ds(off, 16)], r16, mask=m)
                cnt = plsc.all_reduce_population_count(m)
                off = off + cnt[0]
            return off

        return lax.fori_loop(0, EB, _row, off)

    pltpu.async_copy(src_hbm.at[0], seb0, semA)
    pltpu.async_copy(dst_hbm.at[0], deb0, semA)
    pltpu.async_copy(src_hbm.at[1], seb1, semB)
    pltpu.async_copy(dst_hbm.at[1], deb1, semB)

    def _blockpair(q, off):
        b0 = 2 * q
        pltpu.make_async_copy(src_hbm.at[0], seb0, semA).wait()
        pltpu.make_async_copy(dst_hbm.at[0], deb0, semA).wait()
        off = _scan_buf(seb0, deb0, off)

        @pl.when(b0 + 2 < NB - 1)
        def _():
            pltpu.async_copy(src_hbm.at[b0 + 2], seb0, semA)
            pltpu.async_copy(dst_hbm.at[b0 + 2], deb0, semA)

        pltpu.make_async_copy(src_hbm.at[0], seb1, semB).wait()
        pltpu.make_async_copy(dst_hbm.at[0], deb1, semB).wait()
        off = _scan_buf(seb1, deb1, off)

        @pl.when(b0 + 3 < NB - 1)
        def _():
            pltpu.async_copy(src_hbm.at[b0 + 3], seb1, semB)
            pltpu.async_copy(dst_hbm.at[b0 + 3], deb1, semB)

        return off

    m_cnt = lax.fori_loop(0, (NB - 1) // 2, _blockpair, jnp.int32(0))
    pltpu.sync_copy(src_hbm.at[NB - 1], seb0)
    pltpu.sync_copy(dst_hbm.at[NB - 1], deb0)
    m_cnt = _scan_buf(seb0, deb0, m_cnt)

    _prof_scan.__exit__(None, None, None)
    _prof_acc = jax.named_scope("sc_accum")
    _prof_acc.__enter__()
    nq = (m_cnt + CH - 1) // CH
    nqe = jnp.maximum((nq + 1) // 2 * 2, 2)

    lane0f = (jnp.arange(16, dtype=jnp.int32) == 0).astype(jnp.float32)

    def _accum_chunk(g, st):
        for k in range(CH // 16):
            s16 = srcl_v[pl.ds(st + k * 16, 16)]
            r16 = rl_v[pl.ds(st + k * 16, 16)]
            d16 = r16 + lo
            a = (plsc.load_gather(si_v, [d16])
                 + plsc.load_gather(sj_v, [s16]))
            a = jnp.where(a >= 0.0, a, 0.2 * a)
            w16 = jnp.where(s16 == d16, 0.0, jnp.exp(a))
            for t in range(16):
                e = k * 16 + t
                wv = jnp.full((16,), w16[t], jnp.float32)
                r = r16[t]
                for kk in range(C // 16):
                    plsc.addupdate(acc_v.at[r, pl.ds(kk * 16, 16)],
                                   g[e, pl.ds(kk * 16, 16)] * wv)
                plsc.addupdate(den_v.at[pl.ds(r, 16)], wv * lane0f)

    pltpu.async_copy(x_hbm.at[srcl_v.at[pl.ds(0, CH)]], g0, semG0)
    st1 = pl.multiple_of(CH, CH)
    pltpu.async_copy(x_hbm.at[srcl_v.at[pl.ds(st1, CH)]], g1, semG1)

    def _chunkpair(q, carry):
        st0 = pl.multiple_of(2 * q * CH, CH)
        pltpu.make_async_copy(x_hbm.at[pl.ds(0, CH)], g0, semG0).wait()
        _accum_chunk(g0, st0)
        pltpu.make_async_copy(x_hbm.at[pl.ds(0, CH)], g1, semG1).wait()
        _accum_chunk(g1, pl.multiple_of((2 * q + 1) * CH, CH))

        @pl.when(2 * q + 2 < nqe)
        def _():
            st2 = pl.multiple_of((2 * q + 2) * CH, CH)
            pltpu.async_copy(x_hbm.at[srcl_v.at[pl.ds(st2, CH)]], g0, semG0)
            st3 = pl.multiple_of((2 * q + 3) * CH, CH)
            pltpu.async_copy(x_hbm.at[srcl_v.at[pl.ds(st3, CH)]], g1, semG1)

        return carry

    lax.fori_loop(0, nqe // 2, _chunkpair, 0)

    lax.fori_loop(0, 0, lambda i, c: c, 0)
    _prof_acc.__exit__(None, None, None)
    pltpu.sync_copy(acc_v, out_hbm.at[pl.ds(wid * RPT, RPT)])
    pltpu.sync_copy(den_v.at[pl.ds(0, RPT)], den_hbm.at[pl.ds(wid * RPT, RPT)])


def _combine_body(p_ref, d_ref, x_ref, s_ref, g_ref, b_ref, out_ref):
    si = s_ref[:, 0:1]
    sj = s_ref[:, 1:2]
    a = si + sj
    a = jnp.where(a >= 0.0, a, 0.2 * a)
    wself = jnp.exp(a)
    x = x_ref[...]
    num = p_ref[0:N, 0:C] + wself * x
    den = d_ref[0:N, :] + wself
    out = num / jnp.maximum(den, 1e-16)
    mean = jnp.mean(out, axis=0, keepdims=True)
    var = jnp.mean((out - mean) ** 2, axis=0, keepdims=True)
    out = (out - mean) * (g_ref[...] / jnp.sqrt(var + 1e-5)) + b_ref[...]
    out_ref[...] = jnp.maximum(out, 0.0)


def kernel(batch_mat, topk_edge, embedding, W, att_i, att_j, att_em_i,
           att_em_j, bias, gamma, beta):
    del bias
    f32 = jnp.float32
    attm = jnp.zeros((C, C), f32).at[:, 0].set(att_i).at[:, 1].set(att_j)
    attem = (jnp.zeros((C, C), f32).at[:, 0].set(att_em_i)
             .at[:, 1].set(att_em_j))

    x, s = pl.pallas_call(
        _dense_body,
        out_shape=(jax.ShapeDtypeStruct((N, C), f32),
                   jax.ShapeDtypeStruct((N, C), f32)),
    )(batch_mat, embedding, W.T, attm, attem)

    src_r = topk_edge[0].reshape(NB, EB, C)
    dst_r = topk_edge[1].reshape(NB, EB, C)
    si = jnp.pad(s[:, 0], (0, NP - N))
    sj = jnp.pad(s[:, 1], (0, NP - N))

    sc_edges = pl.kernel(
        _sc_body,
        out_type=(jax.ShapeDtypeStruct((NP, C), f32),
                  jax.ShapeDtypeStruct((NP,), f32)),
        mesh=plsc.VectorSubcoreMesh(core_axis_name="c", subcore_axis_name="s"),
        compiler_params=pltpu.CompilerParams(needs_layout_passes=False),
        scratch_types=[
            pltpu.VMEM((EB, C), jnp.int32),
            pltpu.VMEM((EB, C), jnp.int32),
            pltpu.VMEM((EB, C), jnp.int32),
            pltpu.VMEM((EB, C), jnp.int32),
            pltpu.VMEM((NP,), f32),
            pltpu.VMEM((NP,), f32),
            pltpu.VMEM((CAPL,), jnp.int32),
            pltpu.VMEM((CAPL,), jnp.int32),
            pltpu.VMEM((CH, C), f32),
            pltpu.VMEM((CH, C), f32),
            pltpu.VMEM((RPT, C), f32),
            pltpu.VMEM((RPT + 32, ), f32),
            pltpu.SemaphoreType.DMA,
            pltpu.SemaphoreType.DMA,
            pltpu.SemaphoreType.DMA,
            pltpu.SemaphoreType.DMA,
        ],
    )
    p, d = sc_edges(src_r, dst_r, si, sj, x)

    return pl.pallas_call(
        _combine_body,
        out_shape=jax.ShapeDtypeStruct((N, C), f32),
    )(p, d.reshape(NP, 1), x, s, gamma.reshape(1, C), beta.reshape(1, C))


# speedup vs baseline: 1.4606x; 1.4606x over previous
import jax
import jax.numpy as jnp
from jax import lax
from jax.experimental import pallas as pl
from jax.experimental.pallas import tpu as pltpu
from jax.experimental.pallas import tpu_sc as plsc

N = 10000
E = 320000
C = 128
NP = 10240
NC = 2
NS = 16
NW = NC * NS
RPT = NP // NW
CH = 80
EB = 20
NB = E // (EB * C)
CAPL = 12800


def _dense_body(bm_ref, emb_ref, wt_ref, attm_ref, attem_ref, x_ref, s_ref):
    x = jnp.dot(bm_ref[...], wt_ref[...], preferred_element_type=jnp.float32)
    x_ref[...] = x
    s_ref[...] = (
        jnp.dot(x, attm_ref[...], preferred_element_type=jnp.float32)
        + jnp.dot(emb_ref[...], attem_ref[...], preferred_element_type=jnp.float32)
    )


def _sc_body(src_hbm, dst_hbm, si_hbm, sj_hbm, x_hbm, out_hbm, den_hbm,
             seb0, deb0, seb1, deb1, si_v, sj_v, srcl_v, rl_v,
             g0, g1, acc_v, den_v, semA, semB, semG0, semG1):
    c = lax.axis_index("c")
    s = lax.axis_index("s")
    wid = s * NC + c
    lo = wid * RPT

    zv = jnp.zeros((16,), jnp.float32)
    ziv = jnp.zeros((16,), jnp.int32)
    lov = jnp.full((16,), lo, jnp.int32)

    def _zacc(i, carry):
        for k in range(C // 16):
            acc_v[i, pl.ds(k * 16, 16)] = zv
        return carry

    lax.fori_loop(0, RPT, _zacc, 0)
    for k in range((RPT + 32) // 16):
        den_v[pl.ds(k * 16, 16)] = zv

    def _zloc(i, carry):
        srcl_v[pl.ds(i * 16, 16)] = lov
        rl_v[pl.ds(i * 16, 16)] = ziv
        return carry

    lax.fori_loop(0, CAPL // 16, _zloc, 0)

    pltpu.sync_copy(si_hbm, si_v)
    pltpu.sync_copy(sj_hbm, sj_v)

    def _scan_buf(seb, deb, off):
        def _row(i, off):
            for k in range(C // 16):
                s16 = seb[i, pl.ds(k * 16, 16)]
                d16 = deb[i, pl.ds(k * 16, 16)]
                r16 = d16 - lo
                m = (d16 >= lo) & (d16 < lo + RPT)
                plsc.store_compressed(srcl_v.at[pl.ds(off, 16)], s16, mask=m)
                plsc.store_compressed(rl_v.at[pl.ds(off, 16)], r16, mask=m)
                cnt = plsc.all_reduce_population_count(m)
                off = off + cnt[0]
            return off

        return lax.fori_loop(0, EB, _row, off)

    pltpu.async_copy(src_hbm.at[0], seb0, semA)
    pltpu.async_copy(dst_hbm.at[0], deb0, semA)
    pltpu.async_copy(src_hbm.at[1], seb1, semB)
    pltpu.async_copy(dst_hbm.at[1], deb1, semB)

    def _blockpair(q, off):
        b0 = 2 * q
        pltpu.make_async_copy(src_hbm.at[0], seb0, semA).wait()
        pltpu.make_async_copy(dst_hbm.at[0], deb0, semA).wait()
        off = _scan_buf(seb0, deb0, off)

        @pl.when(b0 + 2 < NB - 1)
        def _():
            pltpu.async_copy(src_hbm.at[b0 + 2], seb0, semA)
            pltpu.async_copy(dst_hbm.at[b0 + 2], deb0, semA)

        pltpu.make_async_copy(src_hbm.at[0], seb1, semB).wait()
        pltpu.make_async_copy(dst_hbm.at[0], deb1, semB).wait()
        off = _scan_buf(seb1, deb1, off)

        @pl.when(b0 + 3 < NB - 1)
        def _():
            pltpu.async_copy(src_hbm.at[b0 + 3], seb1, semB)
            pltpu.async_copy(dst_hbm.at[b0 + 3], deb1, semB)

        return off

    m_cnt = lax.fori_loop(0, (NB - 1) // 2, _blockpair, jnp.int32(0))
    pltpu.sync_copy(src_hbm.at[NB - 1], seb0)
    pltpu.sync_copy(dst_hbm.at[NB - 1], deb0)
    m_cnt = _scan_buf(seb0, deb0, m_cnt)

    nq = (m_cnt + CH - 1) // CH
    nqe = jnp.maximum((nq + 1) // 2 * 2, 2)

    lane0f = (jnp.arange(16, dtype=jnp.int32) == 0).astype(jnp.float32)

    def _accum_chunk(g, st):
        for k in range(CH // 16):
            s16 = srcl_v[pl.ds(st + k * 16, 16)]
            r16 = rl_v[pl.ds(st + k * 16, 16)]
            d16 = r16 + lo
            a = (plsc.load_gather(si_v, [d16])
                 + plsc.load_gather(sj_v, [s16]))
            a = jnp.where(a >= 0.0, a, 0.2 * a)
            w16 = jnp.where(s16 == d16, 0.0, jnp.exp(a))
            for t in range(0, 16, 2):
                e0 = k * 16 + t
                e1 = e0 + 1
                wv0 = jnp.full((16,), w16[t], jnp.float32)
                wv1 = jnp.full((16,), w16[t + 1], jnp.float32)
                r0 = r16[t]
                r1 = r16[t + 1]
                vals0 = [g[e0, pl.ds(kk * 16, 16)] * wv0
                         for kk in range(C // 16)]
                vals1 = [g[e1, pl.ds(kk * 16, 16)] * wv1
                         for kk in range(C // 16)]
                for kk in range(C // 16):
                    plsc.addupdate(acc_v.at[r0, pl.ds(kk * 16, 16)],
                                   vals0[kk])
                for kk in range(C // 16):
                    plsc.addupdate(acc_v.at[r1, pl.ds(kk * 16, 16)],
                                   vals1[kk])
                plsc.addupdate(den_v.at[pl.ds(r0, 16)], wv0 * lane0f)
                plsc.addupdate(den_v.at[pl.ds(r1, 16)], wv1 * lane0f)

    pltpu.async_copy(x_hbm.at[srcl_v.at[pl.ds(0, CH)]], g0, semG0)
    st1 = pl.multiple_of(CH, CH)
    pltpu.async_copy(x_hbm.at[srcl_v.at[pl.ds(st1, CH)]], g1, semG1)

    def _chunkpair(q, carry):
        st0 = pl.multiple_of(2 * q * CH, CH)
        pltpu.make_async_copy(x_hbm.at[pl.ds(0, CH)], g0, semG0).wait()
        _accum_chunk(g0, st0)
        pltpu.make_async_copy(x_hbm.at[pl.ds(0, CH)], g1, semG1).wait()
        _accum_chunk(g1, pl.multiple_of((2 * q + 1) * CH, CH))

        @pl.when(2 * q + 2 < nqe)
        def _():
            st2 = pl.multiple_of((2 * q + 2) * CH, CH)
            pltpu.async_copy(x_hbm.at[srcl_v.at[pl.ds(st2, CH)]], g0, semG0)
            st3 = pl.multiple_of((2 * q + 3) * CH, CH)
            pltpu.async_copy(x_hbm.at[srcl_v.at[pl.ds(st3, CH)]], g1, semG1)

        return carry

    lax.fori_loop(0, nqe // 2, _chunkpair, 0)

    pltpu.sync_copy(acc_v, out_hbm.at[pl.ds(wid * RPT, RPT)])
    pltpu.sync_copy(den_v.at[pl.ds(0, RPT)], den_hbm.at[pl.ds(wid * RPT, RPT)])


def _combine_body(p_ref, d_ref, x_ref, s_ref, g_ref, b_ref, out_ref):
    si = s_ref[:, 0:1]
    sj = s_ref[:, 1:2]
    a = si + sj
    a = jnp.where(a >= 0.0, a, 0.2 * a)
    wself = jnp.exp(a)
    x = x_ref[...]
    num = p_ref[0:N, 0:C] + wself * x
    den = d_ref[0:N, :] + wself
    out = num / jnp.maximum(den, 1e-16)
    mean = jnp.mean(out, axis=0, keepdims=True)
    var = jnp.mean((out - mean) ** 2, axis=0, keepdims=True)
    out = (out - mean) * (g_ref[...] / jnp.sqrt(var + 1e-5)) + b_ref[...]
    out_ref[...] = jnp.maximum(out, 0.0)


def kernel(batch_mat, topk_edge, embedding, W, att_i, att_j, att_em_i,
           att_em_j, bias, gamma, beta):
    del bias
    f32 = jnp.float32
    attm = jnp.zeros((C, C), f32).at[:, 0].set(att_i).at[:, 1].set(att_j)
    attem = (jnp.zeros((C, C), f32).at[:, 0].set(att_em_i)
             .at[:, 1].set(att_em_j))

    x, s = pl.pallas_call(
        _dense_body,
        out_shape=(jax.ShapeDtypeStruct((N, C), f32),
                   jax.ShapeDtypeStruct((N, C), f32)),
    )(batch_mat, embedding, W.T, attm, attem)

    src_r = topk_edge[0].reshape(NB, EB, C)
    dst_r = topk_edge[1].reshape(NB, EB, C)
    si = jnp.pad(s[:, 0], (0, NP - N))
    sj = jnp.pad(s[:, 1], (0, NP - N))

    sc_edges = pl.kernel(
        _sc_body,
        out_type=(jax.ShapeDtypeStruct((NP, C), f32),
                  jax.ShapeDtypeStruct((NP,), f32)),
        mesh=plsc.VectorSubcoreMesh(core_axis_name="c", subcore_axis_name="s"),
        compiler_params=pltpu.CompilerParams(needs_layout_passes=False),
        scratch_types=[
            pltpu.VMEM((EB, C), jnp.int32),
            pltpu.VMEM((EB, C), jnp.int32),
            pltpu.VMEM((EB, C), jnp.int32),
            pltpu.VMEM((EB, C), jnp.int32),
            pltpu.VMEM((NP,), f32),
            pltpu.VMEM((NP,), f32),
            pltpu.VMEM((CAPL,), jnp.int32),
            pltpu.VMEM((CAPL,), jnp.int32),
            pltpu.VMEM((CH, C), f32),
            pltpu.VMEM((CH, C), f32),
            pltpu.VMEM((RPT, C), f32),
            pltpu.VMEM((RPT + 32, ), f32),
            pltpu.SemaphoreType.DMA,
            pltpu.SemaphoreType.DMA,
            pltpu.SemaphoreType.DMA,
            pltpu.SemaphoreType.DMA,
        ],
    )
    p, d = sc_edges(src_r, dst_r, si, sj, x)

    return pl.pallas_call(
        _combine_body,
        out_shape=jax.ShapeDtypeStruct((N, C), f32),
    )(p, d.reshape(NP, 1), x, s, gamma.reshape(1, C), beta.reshape(1, C))


# speedup vs baseline: 1.7352x; 1.1880x over previous
import jax
import jax.numpy as jnp
from jax import lax
from jax.experimental import pallas as pl
from jax.experimental.pallas import tpu as pltpu
from jax.experimental.pallas import tpu_sc as plsc

N = 10000
E = 320000
C = 128
NP = 10240
NC = 2
NS = 16
NW = NC * NS
RPT = NP // NW
CH = 80
EB = 20
NB = E // (EB * C)
CAPL = 12800


def _dense_body(bm_ref, emb_ref, wt_ref, attm_ref, attem_ref, x_ref, s_ref):
    x = jnp.dot(bm_ref[...], wt_ref[...], preferred_element_type=jnp.float32)
    x_ref[...] = x
    s_ref[...] = (
        jnp.dot(x, attm_ref[...], preferred_element_type=jnp.float32)
        + jnp.dot(emb_ref[...], attem_ref[...], preferred_element_type=jnp.float32)
    )


def _sc_body(src_hbm, dst_hbm, si_hbm, sj_hbm, x_hbm, out_hbm, den_hbm,
             seb0, deb0, seb1, deb1, si_v, sj_v, srcl_v, rl_v,
             g0, g1, acc_v, den_v, semA, semB, semG0, semG1):
    c = lax.axis_index("c")
    s = lax.axis_index("s")
    wid = s * NC + c
    lo = wid * RPT

    zv = jnp.zeros((16,), jnp.float32)
    ziv = jnp.zeros((16,), jnp.int32)
    lov = jnp.full((16,), lo, jnp.int32)

    def _zacc(i, carry):
        for k in range(C // 16):
            acc_v[i, pl.ds(k * 16, 16)] = zv
        return carry

    lax.fori_loop(0, RPT, _zacc, 0)
    for k in range((RPT + 32) // 16):
        den_v[pl.ds(k * 16, 16)] = zv

    def _zloc(i, carry):
        srcl_v[pl.ds(i * 16, 16)] = lov
        rl_v[pl.ds(i * 16, 16)] = ziv
        return carry

    lax.fori_loop(0, CAPL // 16, _zloc, 0)

    pltpu.sync_copy(si_hbm, si_v)
    pltpu.sync_copy(sj_hbm, sj_v)

    def _scan_buf(seb, deb, off):
        def _row(i, off):
            nk = C // 16
            s_list = [seb[i, pl.ds(k * 16, 16)] for k in range(nk)]
            d_list = [deb[i, pl.ds(k * 16, 16)] for k in range(nk)]
            r_list = [d - lo for d in d_list]
            m_list = [plsc.bitcast(r, jnp.uint32) < jnp.uint32(RPT)
                      for r in r_list]
            cnts = [plsc.all_reduce_population_count(m)[0] for m in m_list]
            offs = []
            for k in range(nk):
                offs.append(off)
                off = off + cnts[k]
            for k in range(nk):
                plsc.store_compressed(srcl_v.at[pl.ds(offs[k], 16)],
                                      s_list[k], mask=m_list[k])
                plsc.store_compressed(rl_v.at[pl.ds(offs[k], 16)],
                                      r_list[k], mask=m_list[k])
            return off

        return lax.fori_loop(0, EB, _row, off)

    pltpu.async_copy(src_hbm.at[0], seb0, semA)
    pltpu.async_copy(dst_hbm.at[0], deb0, semA)
    pltpu.async_copy(src_hbm.at[1], seb1, semB)
    pltpu.async_copy(dst_hbm.at[1], deb1, semB)

    def _blockpair(q, off):
        b0 = 2 * q
        pltpu.make_async_copy(src_hbm.at[0], seb0, semA).wait()
        pltpu.make_async_copy(dst_hbm.at[0], deb0, semA).wait()
        off = _scan_buf(seb0, deb0, off)

        @pl.when(b0 + 2 < NB - 1)
        def _():
            pltpu.async_copy(src_hbm.at[b0 + 2], seb0, semA)
            pltpu.async_copy(dst_hbm.at[b0 + 2], deb0, semA)

        pltpu.make_async_copy(src_hbm.at[0], seb1, semB).wait()
        pltpu.make_async_copy(dst_hbm.at[0], deb1, semB).wait()
        off = _scan_buf(seb1, deb1, off)

        @pl.when(b0 + 3 < NB - 1)
        def _():
            pltpu.async_copy(src_hbm.at[b0 + 3], seb1, semB)
            pltpu.async_copy(dst_hbm.at[b0 + 3], deb1, semB)

        return off

    m_cnt = lax.fori_loop(0, (NB - 1) // 2, _blockpair, jnp.int32(0))
    pltpu.sync_copy(src_hbm.at[NB - 1], seb0)
    pltpu.sync_copy(dst_hbm.at[NB - 1], deb0)
    m_cnt = _scan_buf(seb0, deb0, m_cnt)

    nq = (m_cnt + CH - 1) // CH
    nqe = jnp.maximum((nq + 1) // 2 * 2, 2)

    lane0f = (jnp.arange(16, dtype=jnp.int32) == 0).astype(jnp.float32)

    def _accum_chunk(g, st):
        for k in range(CH // 16):
            s16 = srcl_v[pl.ds(st + k * 16, 16)]
            r16 = rl_v[pl.ds(st + k * 16, 16)]
            d16 = r16 + lo
            a = (plsc.load_gather(si_v, [d16])
                 + plsc.load_gather(sj_v, [s16]))
            a = jnp.where(a >= 0.0, a, 0.2 * a)
            w16 = jnp.where(s16 == d16, 0.0, jnp.exp(a))
            for t in range(0, 16, 2):
                e0 = k * 16 + t
                e1 = e0 + 1
                wv0 = jnp.full((16,), w16[t], jnp.float32)
                wv1 = jnp.full((16,), w16[t + 1], jnp.float32)
                r0 = r16[t]
                r1 = r16[t + 1]
                vals0 = [g[e0, pl.ds(kk * 16, 16)] * wv0
                         for kk in range(C // 16)]
                vals1 = [g[e1, pl.ds(kk * 16, 16)] * wv1
                         for kk in range(C // 16)]
                for kk in range(C // 16):
                    plsc.addupdate(acc_v.at[r0, pl.ds(kk * 16, 16)],
                                   vals0[kk])
                for kk in range(C // 16):
                    plsc.addupdate(acc_v.at[r1, pl.ds(kk * 16, 16)],
                                   vals1[kk])
                plsc.addupdate(den_v.at[pl.ds(r0, 16)], wv0 * lane0f)
                plsc.addupdate(den_v.at[pl.ds(r1, 16)], wv1 * lane0f)

    pltpu.async_copy(x_hbm.at[srcl_v.at[pl.ds(0, CH)]], g0, semG0)
    st1 = pl.multiple_of(CH, CH)
    pltpu.async_copy(x_hbm.at[srcl_v.at[pl.ds(st1, CH)]], g1, semG1)

    def _chunkpair(q, carry):
        st0 = pl.multiple_of(2 * q * CH, CH)
        pltpu.make_async_copy(x_hbm.at[pl.ds(0, CH)], g0, semG0).wait()
        _accum_chunk(g0, st0)
        pltpu.make_async_copy(x_hbm.at[pl.ds(0, CH)], g1, semG1).wait()
        _accum_chunk(g1, pl.multiple_of((2 * q + 1) * CH, CH))

        @pl.when(2 * q + 2 < nqe)
        def _():
            st2 = pl.multiple_of((2 * q + 2) * CH, CH)
            pltpu.async_copy(x_hbm.at[srcl_v.at[pl.ds(st2, CH)]], g0, semG0)
            st3 = pl.multiple_of((2 * q + 3) * CH, CH)
            pltpu.async_copy(x_hbm.at[srcl_v.at[pl.ds(st3, CH)]], g1, semG1)

        return carry

    lax.fori_loop(0, nqe // 2, _chunkpair, 0)

    pltpu.sync_copy(acc_v, out_hbm.at[pl.ds(wid * RPT, RPT)])
    pltpu.sync_copy(den_v.at[pl.ds(0, RPT)], den_hbm.at[pl.ds(wid * RPT, RPT)])


def _combine_body(p_ref, d_ref, x_ref, s_ref, g_ref, b_ref, out_ref):
    si = s_ref[:, 0:1]
    sj = s_ref[:, 1:2]
    a = si + sj
    a = jnp.where(a >= 0.0, a, 0.2 * a)
    wself = jnp.exp(a)
    x = x_ref[...]
    num = p_ref[0:N, 0:C] + wself * x
    den = d_ref[0:N, :] + wself
    out = num / jnp.maximum(den, 1e-16)
    mean = jnp.mean(out, axis=0, keepdims=True)
    var = jnp.mean((out - mean) ** 2, axis=0, keepdims=True)
    out = (out - mean) * (g_ref[...] / jnp.sqrt(var + 1e-5)) + b_ref[...]
    out_ref[...] = jnp.maximum(out, 0.0)


def kernel(batch_mat, topk_edge, embedding, W, att_i, att_j, att_em_i,
           att_em_j, bias, gamma, beta):
    del bias
    f32 = jnp.float32
    attm = jnp.zeros((C, C), f32).at[:, 0].set(att_i).at[:, 1].set(att_j)
    attem = (jnp.zeros((C, C), f32).at[:, 0].set(att_em_i)
             .at[:, 1].set(att_em_j))

    x, s = pl.pallas_call(
        _dense_body,
        out_shape=(jax.ShapeDtypeStruct((N, C), f32),
                   jax.ShapeDtypeStruct((N, C), f32)),
    )(batch_mat, embedding, W.T, attm, attem)

    src_r = topk_edge[0].reshape(NB, EB, C)
    dst_r = topk_edge[1].reshape(NB, EB, C)
    si = jnp.pad(s[:, 0], (0, NP - N))
    sj = jnp.pad(s[:, 1], (0, NP - N))

    sc_edges = pl.kernel(
        _sc_body,
        out_type=(jax.ShapeDtypeStruct((NP, C), f32),
                  jax.ShapeDtypeStruct((NP,), f32)),
        mesh=plsc.VectorSubcoreMesh(core_axis_name="c", subcore_axis_name="s"),
        compiler_params=pltpu.CompilerParams(needs_layout_passes=False),
        scratch_types=[
            pltpu.VMEM((EB, C), jnp.int32),
            pltpu.VMEM((EB, C), jnp.int32),
            pltpu.VMEM((EB, C), jnp.int32),
            pltpu.VMEM((EB, C), jnp.int32),
            pltpu.VMEM((NP,), f32),
            pltpu.VMEM((NP,), f32),
            pltpu.VMEM((CAPL,), jnp.int32),
            pltpu.VMEM((CAPL,), jnp.int32),
            pltpu.VMEM((CH, C), f32),
            pltpu.VMEM((CH, C), f32),
            pltpu.VMEM((RPT, C), f32),
            pltpu.VMEM((RPT + 32, ), f32),
            pltpu.SemaphoreType.DMA,
            pltpu.SemaphoreType.DMA,
            pltpu.SemaphoreType.DMA,
            pltpu.SemaphoreType.DMA,
        ],
    )
    p, d = sc_edges(src_r, dst_r, si, sj, x)

    return pl.pallas_call(
        _combine_body,
        out_shape=jax.ShapeDtypeStruct((N, C), f32),
    )(p, d.reshape(NP, 1), x, s, gamma.reshape(1, C), beta.reshape(1, C))
